# baseline (speedup 1.0000x reference)
import jax
import jax.numpy as jnp
from jax.experimental import pallas as pl

CUTOFF = 10.0
N_RBF = 8
N_LAYERS = 3
N_GRAPHS = 1024


def _readout_body(h_ref, wl_ref, bl_ref, out_ref):
    out_ref[...] = h_ref[...] @ wl_ref[...] + bl_ref[0]


def kernel(pos, z_idx, batch, edge_index, Wm, bm, Wu, bu, Wl, bl):
    N = pos.shape[0]
    one_hot = jax.nn.one_hot(z_idx, 5, dtype=jnp.float32)
    h = jnp.concatenate([one_hot, jnp.zeros((N, 1), jnp.float32)], axis=1) - 0.5
    h = jnp.concatenate([h, jnp.ones((N, 1), jnp.float32), jnp.zeros((N, 1), jnp.float32)], axis=1)
    src, dst = edge_index[0], edge_index[1]
    rel = pos[src] - pos[dst]
    d = jnp.sqrt(jnp.sum(rel * rel, axis=-1, keepdims=True) + 1e-12)
    centers = jnp.linspace(0.0, CUTOFF, N_RBF).reshape(1, N_RBF)
    gamma = (CUTOFF / N_RBF) ** 2
    rbf = jnp.exp(-((d - centers) ** 2) / gamma)
    env = 0.5 * (jnp.cos(jnp.pi * jnp.clip(d / CUTOFF, 0.0, 1.0)) + 1.0)
    rbf = rbf * env
    for l in range(N_LAYERS):
        m_in = jnp.concatenate([h[src], h[dst], rbf], axis=1)
        m = jax.nn.silu(m_in @ Wm[l] + bm[l])
        agg = jax.ops.segment_sum(m, dst, num_segments=N)
        u = jnp.concatenate([h, agg], axis=1)
        h = h + jax.nn.silu(u @ Wu[l] + bu[l])

    BL = 2000
    out = pl.pallas_call(
        _readout_body,
        grid=(N // BL,),
        in_specs=[
            pl.BlockSpec((BL, 8), lambda i: (i, 0)),
            pl.BlockSpec((8, 1), lambda i: (0, 0)),
            pl.BlockSpec((1,), lambda i: (0,)),
        ],
        out_specs=pl.BlockSpec((BL, 1), lambda i: (i, 0)),
        out_shape=jax.ShapeDtypeStruct((N, 1), jnp.float32),
    )(h, Wl, bl)
    return jax.ops.segment_sum(out, batch, num_segments=N_GRAPHS)


# speedup vs baseline: 3.0067x; 3.0067x over previous
import functools
import math

import jax
import jax.numpy as jnp
from jax import lax
from jax.experimental import pallas as pl
from jax.experimental.pallas import tpu as pltpu
from jax.experimental.pallas import tpu_sc as plsc

CUTOFF = 10.0
N_RBF = 8
N_LAYERS = 3
N_GRAPHS = 1024
N_NODES = 50000
N_EDGES = 1600000

NTILES = 16
NPAD = 50176
NODES_PER_TILE = NPAD // NTILES
NCHUNK = 784
ECHUNK = 400
EDGES_PER_TILE = N_EDGES // NTILES
GPAD = 1152
GPT = N_GRAPHS // NTILES
GAMMA = (CUTOFF / N_RBF) ** 2

WM_OFF = 0
BM_OFF = WM_OFF + N_LAYERS * 24 * 8
WU_OFF = BM_OFF + N_LAYERS * 8
BU_OFF = WU_OFF + N_LAYERS * 16 * 8
WL_OFF = BU_OFF + N_LAYERS * 8
BL_OFF = WL_OFF + 8


def _silu(x):
  return x / (1.0 + jnp.exp(-x))


def _body(h0_hbm, pos_hbm, src_hbm, dst_hbm, batch_hbm, w_hbm, z_hbm, one_hbm,
          out_hbm,
          h_sh, pos_sh, agg_sh, hsum_sh,
          sidx_v, didx_v, gs, gd, mc, hc, ac, ho, bidx_v, ocol, zv, w_v, rc,
          out_v, sem1, sem2):
  tid = lax.axis_index("s")
  nb_t = tid * NODES_PER_TILE

  def hj(ref, j):
    return ref.at[pl.ds(j * NPAD, NPAD)]

  stage = hc.at[pl.ds(0, NODES_PER_TILE)]
  for j in range(8):
    sl = pl.ds(j * NPAD + nb_t, NODES_PER_TILE)
    pltpu.sync_copy(h0_hbm.at[sl], stage)
    pltpu.sync_copy(stage, h_sh.at[sl])
  for a in range(3):
    sl = pl.ds(a * NPAD + nb_t, NODES_PER_TILE)
    pltpu.sync_copy(pos_hbm.at[sl], stage)
    pltpu.sync_copy(stage, pos_sh.at[sl])
  pltpu.sync_copy(z_hbm.at[pl.ds(0, NCHUNK)], zv)
  for j in range(8):
    for c in range(NODES_PER_TILE // NCHUNK):
      pltpu.sync_copy(zv, agg_sh.at[pl.ds(j * NPAD + nb_t + c * NCHUNK,
                                          NCHUNK)])
  gz = GPAD // NTILES
  for j in range(9):
    pltpu.sync_copy(zv.at[pl.ds(0, gz)],
                    hsum_sh.at[pl.ds(j * GPAD + tid * gz, gz)])
  pltpu.sync_copy(w_hbm, w_v)
  pltpu.sync_copy(one_hbm.at[pl.ds(0, NCHUNK)], ocol)
  plsc.subcore_barrier()

  centers = [CUTOFF * k / (N_RBF - 1) for k in range(N_RBF)]

  def _wscal(off):
    return w_v[pl.ds((off // 16) * 16, 16)][off % 16]

  for l in range(N_LAYERS):
    wm = [_wscal(WM_OFF + l * 192 + i) for i in range(192)]
    bm_s = [_wscal(BM_OFF + l * 8 + j) for j in range(8)]
    wu = [_wscal(WU_OFF + l * 128 + i) for i in range(128)]
    bu_s = [_wscal(BU_OFF + l * 8 + j) for j in range(8)]

    def edge_chunk(c, _):
      base = tid * EDGES_PER_TILE + c * ECHUNK
      pltpu.sync_copy(src_hbm.at[pl.ds(base, ECHUNK)], sidx_v)
      pltpu.sync_copy(dst_hbm.at[pl.ds(base, ECHUNK)], didx_v)
      cps = []
      for j in range(8):
        cps.append(pltpu.async_copy(hj(h_sh, j).at[sidx_v],
                                    gs.at[pl.ds(j * ECHUNK, ECHUNK)], sem1))
        cps.append(pltpu.async_copy(hj(h_sh, j).at[didx_v],
                                    gd.at[pl.ds(j * ECHUNK, ECHUNK)], sem2))
      for a in range(3):
        cps.append(pltpu.async_copy(
            pos_sh.at[pl.ds(a * NPAD, NPAD)].at[sidx_v],
            gs.at[pl.ds((8 + a) * ECHUNK, ECHUNK)], sem1))
        cps.append(pltpu.async_copy(
            pos_sh.at[pl.ds(a * NPAD, NPAD)].at[didx_v],
            gd.at[pl.ds((8 + a) * ECHUNK, ECHUNK)], sem2))
      for cp in cps:
        cp.wait()

      def group(g, _):
        o = g * 16
        hs = [gs[pl.ds(j * ECHUNK + o, 16)] for j in range(8)]
        hd = [gd[pl.ds(j * ECHUNK + o, 16)] for j in range(8)]
        rel = [gs[pl.ds((8 + a) * ECHUNK + o, 16)]
               - gd[pl.ds((8 + a) * ECHUNK + o, 16)] for a in range(3)]
        r2 = rel[0] * rel[0] + rel[1] * rel[1] + rel[2] * rel[2] + 1e-12
        i = lax.bitcast_convert_type(r2, jnp.int32)
        i = jnp.int32(0x5F3759DF) - lax.shift_right_logical(i, 1)
        y = lax.bitcast_convert_type(i, jnp.float32)
        half = 0.5 * r2
        for _n in range(3):
          y = y * (1.5 - half * y * y)
        d = r2 * y
        x = jnp.minimum(jnp.maximum(d * (1.0 / CUTOFF), 0.0), 1.0)
        u = (0.5 - x) * math.pi
        u2 = u * u
        s = u * (1.0 + u2 * (-1.0 / 6.0 + u2 * (1.0 / 120.0 + u2 * (
            -1.0 / 5040.0 + u2 * (1.0 / 362880.0 + u2 * (-1.0 / 39916800.0))))))
        env = 0.5 * (s + 1.0)
        rbf = []
        for k in range(N_RBF):
          t = d - centers[k]
          rbf.append(jnp.exp(t * t * (-1.0 / GAMMA)) * env)
        feats = hs + hd + rbf
        for j in range(8):
          acc = feats[0] * wm[j]
          for k in range(1, 24):
            acc = acc + feats[k] * wm[k * 8 + j]
          mc[pl.ds(j * ECHUNK + o, 16)] = _silu(acc + bm_s[j])
        return 0

      lax.fori_loop(0, ECHUNK // 16, group, 0)
      scs = [pltpu.async_copy(mc.at[pl.ds(j * ECHUNK, ECHUNK)],
                              hj(agg_sh, j).at[didx_v], sem2, add=True)
             for j in range(8)]
      for cp in scs:
        cp.wait()
      return 0

    lax.fori_loop(0, EDGES_PER_TILE // ECHUNK, edge_chunk, 0)
    plsc.subcore_barrier()

    last = l == N_LAYERS - 1

    def node_chunk(c, _):
      nbase = nb_t + c * NCHUNK
      cps = []
      for j in range(8):
        cps.append(pltpu.async_copy(h_sh.at[pl.ds(j * NPAD + nbase, NCHUNK)],
                                    hc.at[pl.ds(j * NCHUNK, NCHUNK)], sem1))
        cps.append(pltpu.async_copy(agg_sh.at[pl.ds(j * NPAD + nbase, NCHUNK)],
                                    ac.at[pl.ds(j * NCHUNK, NCHUNK)], sem2))
      for cp in cps:
        cp.wait()

      def group(g, _):
        o = g * 16
        hv = [hc[pl.ds(j * NCHUNK + o, 16)] for j in range(8)]
        av = [ac[pl.ds(j * NCHUNK + o, 16)] for j in range(8)]
        for j in range(8):
          acc = hv[0] * wu[j]
          for k in range(1, 8):
            acc = acc + hv[k] * wu[k * 8 + j]
          for k in range(8):
            acc = acc + av[k] * wu[(8 + k) * 8 + j]
          ho[pl.ds(j * NCHUNK + o, 16)] = hv[j] + _silu(acc + bu_s[j])
        return 0

      lax.fori_loop(0, NCHUNK // 16, group, 0)
      if not last:
        for j in range(8):
          pltpu.sync_copy(ho.at[pl.ds(j * NCHUNK, NCHUNK)],
                          h_sh.at[pl.ds(j * NPAD + nbase, NCHUNK)])
          pltpu.sync_copy(zv, agg_sh.at[pl.ds(j * NPAD + nbase, NCHUNK)])
      else:
        pltpu.sync_copy(batch_hbm.at[pl.ds(nbase, NCHUNK)], bidx_v)
        scs = [pltpu.async_copy(
            ho.at[pl.ds(j * NCHUNK, NCHUNK)],
            hsum_sh.at[pl.ds(j * GPAD, GPAD)].at[bidx_v], sem1, add=True)
            for j in range(8)]
        scs.append(pltpu.async_copy(
            ocol, hsum_sh.at[pl.ds(8 * GPAD, GPAD)].at[bidx_v], sem1,
            add=True))
        for cp in scs:
          cp.wait()
      return 0

    lax.fori_loop(0, NODES_PER_TILE // NCHUNK, node_chunk, 0)
    plsc.subcore_barrier()

  for j in range(9):
    pltpu.sync_copy(hsum_sh.at[pl.ds(j * GPAD + tid * GPT, GPT)],
                    rc.at[pl.ds(j * GPT, GPT)])
  wl = [_wscal(WL_OFF + j) for j in range(8)]
  bl_s = _wscal(BL_OFF)

  def rgroup(g, _):
    o = g * 16
    acc = rc[pl.ds(8 * GPT + o, 16)] * bl_s
    for j in range(8):
      acc = acc + rc[pl.ds(j * GPT + o, 16)] * wl[j]
    out_v[pl.ds(o, 16)] = acc
    return 0

  lax.fori_loop(0, GPT // 16, rgroup, 0)
  pltpu.sync_copy(out_v, out_hbm.at[pl.ds(tid * GPT, GPT)])


@jax.jit
def _run(h0t, post, src, dst, batchp, wflat, zeros1, ones1):
  mesh = plsc.VectorSubcoreMesh(core_axis_name="c", subcore_axis_name="s",
                                num_cores=1)
  f = functools.partial(
      pl.kernel, mesh=mesh,
      out_type=jax.ShapeDtypeStruct((N_GRAPHS,), jnp.float32),
      scratch_types=[
          pltpu.VMEM_SHARED((8 * NPAD,), jnp.float32),
          pltpu.VMEM_SHARED((3 * NPAD,), jnp.float32),
          pltpu.VMEM_SHARED((8 * NPAD,), jnp.float32),
          pltpu.VMEM_SHARED((9 * GPAD,), jnp.float32),
          pltpu.VMEM((ECHUNK,), jnp.int32),
          pltpu.VMEM((ECHUNK,), jnp.int32),
          pltpu.VMEM((11 * ECHUNK,), jnp.float32),
          pltpu.VMEM((11 * ECHUNK,), jnp.float32),
          pltpu.VMEM((8 * ECHUNK,), jnp.float32),
          pltpu.VMEM((8 * NCHUNK,), jnp.float32),
          pltpu.VMEM((8 * NCHUNK,), jnp.float32),
          pltpu.VMEM((8 * NCHUNK,), jnp.float32),
          pltpu.VMEM((NCHUNK,), jnp.int32),
          pltpu.VMEM((NCHUNK,), jnp.float32),
          pltpu.VMEM((NCHUNK,), jnp.float32),
          pltpu.VMEM((1024,), jnp.float32),
          pltpu.VMEM((9 * GPT,), jnp.float32),
          pltpu.VMEM((GPT,), jnp.float32),
          pltpu.SemaphoreType.DMA,
          pltpu.SemaphoreType.DMA,
      ])(_body)
  return f(h0t, post, src, dst, batchp, wflat, zeros1, ones1)


def kernel(pos, z_idx, batch, edge_index, Wm, bm, Wu, bu, Wl, bl):
  N = pos.shape[0]
  one_hot = jax.nn.one_hot(z_idx, 5, dtype=jnp.float32)
  h0 = jnp.concatenate([one_hot, jnp.zeros((N, 1), jnp.float32)], axis=1) - 0.5
  h0 = jnp.concatenate(
      [h0, jnp.ones((N, 1), jnp.float32), jnp.zeros((N, 1), jnp.float32)],
      axis=1)
  h0t = jnp.zeros((8, NPAD), jnp.float32).at[:, :N].set(h0.T).reshape(-1)
  post = jnp.zeros((3, NPAD), jnp.float32).at[:, :N].set(pos.T).reshape(-1)
  src = edge_index[0].astype(jnp.int32)
  dst = edge_index[1].astype(jnp.int32)
  batchp = jnp.full((NPAD,), N_GRAPHS, jnp.int32).at[:N].set(
      batch.astype(jnp.int32))
  wflat = jnp.zeros((1024,), jnp.float32)
  wflat = wflat.at[WM_OFF:WM_OFF + 576].set(Wm.reshape(-1))
  wflat = wflat.at[BM_OFF:BM_OFF + 24].set(bm.reshape(-1))
  wflat = wflat.at[WU_OFF:WU_OFF + 384].set(Wu.reshape(-1))
  wflat = wflat.at[BU_OFF:BU_OFF + 24].set(bu.reshape(-1))
  wflat = wflat.at[WL_OFF:WL_OFF + 8].set(Wl.reshape(-1))
  wflat = wflat.at[BL_OFF].set(bl[0])
  zeros1 = jnp.zeros((NPAD,), jnp.float32)
  ones1 = jnp.ones((NPAD,), jnp.float32)
  out = _run(h0t, post, src, dst, batchp, wflat, zeros1, ones1)
  return out.reshape(N_GRAPHS, 1)


# speedup vs baseline: 3.1712x; 1.0547x over previous
import functools
import math

import jax
import jax.numpy as jnp
from jax import lax
from jax.experimental import pallas as pl
from jax.experimental.pallas import tpu as pltpu
from jax.experimental.pallas import tpu_sc as plsc

CUTOFF = 10.0
N_RBF = 8
N_LAYERS = 3
N_GRAPHS = 1024
N_NODES = 50000
N_EDGES = 1600000

NTILES = 16
NPAD = 50176
NODES_PER_TILE = NPAD // NTILES
NCHUNK = 784
ECHUNK = 800
EDGES_PER_TILE = N_EDGES // NTILES
GPAD = 1152
GPT = N_GRAPHS // NTILES
GAMMA = (CUTOFF / N_RBF) ** 2

WM_OFF = 0
BM_OFF = WM_OFF + N_LAYERS * 24 * 8
WU_OFF = BM_OFF + N_LAYERS * 8
BU_OFF = WU_OFF + N_LAYERS * 16 * 8
WL_OFF = BU_OFF + N_LAYERS * 8
BL_OFF = WL_OFF + 8


def _silu(x):
  return x / (1.0 + jnp.exp(-x))


def _body(h0_hbm, pos_hbm, src_hbm, dst_hbm, batch_hbm, w_hbm, z_hbm, one_hbm,
          out_hbm,
          h_sh, pos_sh, agg_sh, hsum_sh,
          sidx_v, didx_v, gs, gd, mc, hc, ac, ho, bidx_v, ocol, zv, w_v, rc,
          out_v, sem1, sem2):
  tid = lax.axis_index("s")
  nb_t = tid * NODES_PER_TILE

  def hj(ref, j):
    return ref.at[pl.ds(j * NPAD, NPAD)]

  stage = hc.at[pl.ds(0, NODES_PER_TILE)]
  for j in range(8):
    sl = pl.ds(j * NPAD + nb_t, NODES_PER_TILE)
    pltpu.sync_copy(h0_hbm.at[sl], stage)
    pltpu.sync_copy(stage, h_sh.at[sl])
  for a in range(3):
    sl = pl.ds(a * NPAD + nb_t, NODES_PER_TILE)
    pltpu.sync_copy(pos_hbm.at[sl], stage)
    pltpu.sync_copy(stage, pos_sh.at[sl])
  pltpu.sync_copy(z_hbm.at[pl.ds(0, NCHUNK)], zv)
  for j in range(8):
    for c in range(NODES_PER_TILE // NCHUNK):
      pltpu.sync_copy(zv, agg_sh.at[pl.ds(j * NPAD + nb_t + c * NCHUNK,
                                          NCHUNK)])
  gz = GPAD // NTILES
  for j in range(9):
    pltpu.sync_copy(zv.at[pl.ds(0, gz)],
                    hsum_sh.at[pl.ds(j * GPAD + tid * gz, gz)])
  pltpu.sync_copy(w_hbm, w_v)
  pltpu.sync_copy(one_hbm.at[pl.ds(0, NCHUNK)], ocol)
  plsc.subcore_barrier()

  centers = [CUTOFF * k / (N_RBF - 1) for k in range(N_RBF)]

  def _wscal(off):
    return w_v[pl.ds((off // 16) * 16, 16)][off % 16]

  for l in range(N_LAYERS):
    wm = [_wscal(WM_OFF + l * 192 + i) for i in range(192)]
    bm_s = [_wscal(BM_OFF + l * 8 + j) for j in range(8)]
    wu = [_wscal(WU_OFF + l * 128 + i) for i in range(128)]
    bu_s = [_wscal(BU_OFF + l * 8 + j) for j in range(8)]

    def edge_chunk(c, _):
      base = tid * EDGES_PER_TILE + c * ECHUNK
      pltpu.sync_copy(src_hbm.at[pl.ds(base, ECHUNK)], sidx_v)
      pltpu.sync_copy(dst_hbm.at[pl.ds(base, ECHUNK)], didx_v)
      cps = []
      for j in range(8):
        cps.append(pltpu.async_copy(hj(h_sh, j).at[sidx_v],
                                    gs.at[pl.ds(j * ECHUNK, ECHUNK)], sem1))
        cps.append(pltpu.async_copy(hj(h_sh, j).at[didx_v],
                                    gd.at[pl.ds(j * ECHUNK, ECHUNK)], sem2))
      for a in range(3):
        cps.append(pltpu.async_copy(
            pos_sh.at[pl.ds(a * NPAD, NPAD)].at[sidx_v],
            gs.at[pl.ds((8 + a) * ECHUNK, ECHUNK)], sem1))
        cps.append(pltpu.async_copy(
            pos_sh.at[pl.ds(a * NPAD, NPAD)].at[didx_v],
            gd.at[pl.ds((8 + a) * ECHUNK, ECHUNK)], sem2))
      for cp in cps:
        cp.wait()

      def group(g, _):
        o = g * 16
        hs = [gs[pl.ds(j * ECHUNK + o, 16)] for j in range(8)]
        hd = [gd[pl.ds(j * ECHUNK + o, 16)] for j in range(8)]
        rel = [gs[pl.ds((8 + a) * ECHUNK + o, 16)]
               - gd[pl.ds((8 + a) * ECHUNK + o, 16)] for a in range(3)]
        r2 = rel[0] * rel[0] + rel[1] * rel[1] + rel[2] * rel[2] + 1e-12
        i = lax.bitcast_convert_type(r2, jnp.int32)
        i = jnp.int32(0x5F3759DF) - lax.shift_right_logical(i, 1)
        y = lax.bitcast_convert_type(i, jnp.float32)
        half = 0.5 * r2
        for _n in range(3):
          y = y * (1.5 - half * y * y)
        d = r2 * y
        x = jnp.minimum(jnp.maximum(d * (1.0 / CUTOFF), 0.0), 1.0)
        u = (0.5 - x) * math.pi
        u2 = u * u
        s = u * (1.0 + u2 * (-1.0 / 6.0 + u2 * (1.0 / 120.0 + u2 * (
            -1.0 / 5040.0 + u2 * (1.0 / 362880.0 + u2 * (-1.0 / 39916800.0))))))
        env = 0.5 * (s + 1.0)
        rbf = []
        for k in range(N_RBF):
          t = d - centers[k]
          rbf.append(jnp.exp(t * t * (-1.0 / GAMMA)) * env)
        feats = hs + hd + rbf
        for j in range(8):
          acc = feats[0] * wm[j]
          for k in range(1, 24):
            acc = acc + feats[k] * wm[k * 8 + j]
          mc[pl.ds(j * ECHUNK + o, 16)] = _silu(acc + bm_s[j])
        return 0

      lax.fori_loop(0, ECHUNK // 16, group, 0)
      scs = [pltpu.async_copy(mc.at[pl.ds(j * ECHUNK, ECHUNK)],
                              hj(agg_sh, j).at[didx_v], sem2, add=True)
             for j in range(8)]
      for cp in scs:
        cp.wait()
      return 0

    lax.fori_loop(0, EDGES_PER_TILE // ECHUNK, edge_chunk, 0)
    plsc.subcore_barrier()

    last = l == N_LAYERS - 1

    def node_chunk(c, _):
      nbase = nb_t + c * NCHUNK
      cps = []
      for j in range(8):
        cps.append(pltpu.async_copy(h_sh.at[pl.ds(j * NPAD + nbase, NCHUNK)],
                                    hc.at[pl.ds(j * NCHUNK, NCHUNK)], sem1))
        cps.append(pltpu.async_copy(agg_sh.at[pl.ds(j * NPAD + nbase, NCHUNK)],
                                    ac.at[pl.ds(j * NCHUNK, NCHUNK)], sem2))
      for cp in cps:
        cp.wait()

      def group(g, _):
        o = g * 16
        hv = [hc[pl.ds(j * NCHUNK + o, 16)] for j in range(8)]
        av = [ac[pl.ds(j * NCHUNK + o, 16)] for j in range(8)]
        for j in range(8):
          acc = hv[0] * wu[j]
          for k in range(1, 8):
            acc = acc + hv[k] * wu[k * 8 + j]
          for k in range(8):
            acc = acc + av[k] * wu[(8 + k) * 8 + j]
          ho[pl.ds(j * NCHUNK + o, 16)] = hv[j] + _silu(acc + bu_s[j])
        return 0

      lax.fori_loop(0, NCHUNK // 16, group, 0)
      if not last:
        for j in range(8):
          pltpu.sync_copy(ho.at[pl.ds(j * NCHUNK, NCHUNK)],
                          h_sh.at[pl.ds(j * NPAD + nbase, NCHUNK)])
          pltpu.sync_copy(zv, agg_sh.at[pl.ds(j * NPAD + nbase, NCHUNK)])
      else:
        pltpu.sync_copy(batch_hbm.at[pl.ds(nbase, NCHUNK)], bidx_v)
        scs = [pltpu.async_copy(
            ho.at[pl.ds(j * NCHUNK, NCHUNK)],
            hsum_sh.at[pl.ds(j * GPAD, GPAD)].at[bidx_v], sem1, add=True)
            for j in range(8)]
        scs.append(pltpu.async_copy(
            ocol, hsum_sh.at[pl.ds(8 * GPAD, GPAD)].at[bidx_v], sem1,
            add=True))
        for cp in scs:
          cp.wait()
      return 0

    lax.fori_loop(0, NODES_PER_TILE // NCHUNK, node_chunk, 0)
    plsc.subcore_barrier()

  for j in range(9):
    pltpu.sync_copy(hsum_sh.at[pl.ds(j * GPAD + tid * GPT, GPT)],
                    rc.at[pl.ds(j * GPT, GPT)])
  wl = [_wscal(WL_OFF + j) for j in range(8)]
  bl_s = _wscal(BL_OFF)

  def rgroup(g, _):
    o = g * 16
    acc = rc[pl.ds(8 * GPT + o, 16)] * bl_s
    for j in range(8):
      acc = acc + rc[pl.ds(j * GPT + o, 16)] * wl[j]
    out_v[pl.ds(o, 16)] = acc
    return 0

  lax.fori_loop(0, GPT // 16, rgroup, 0)
  pltpu.sync_copy(out_v, out_hbm.at[pl.ds(tid * GPT, GPT)])


@jax.jit
def _run(h0t, post, src, dst, batchp, wflat, zeros1, ones1):
  mesh = plsc.VectorSubcoreMesh(core_axis_name="c", subcore_axis_name="s",
                                num_cores=1)
  f = functools.partial(
      pl.kernel, mesh=mesh,
      out_type=jax.ShapeDtypeStruct((N_GRAPHS,), jnp.float32),
      scratch_types=[
          pltpu.VMEM_SHARED((8 * NPAD,), jnp.float32),
          pltpu.VMEM_SHARED((3 * NPAD,), jnp.float32),
          pltpu.VMEM_SHARED((8 * NPAD,), jnp.float32),
          pltpu.VMEM_SHARED((9 * GPAD,), jnp.float32),
          pltpu.VMEM((ECHUNK,), jnp.int32),
          pltpu.VMEM((ECHUNK,), jnp.int32),
          pltpu.VMEM((11 * ECHUNK,), jnp.float32),
          pltpu.VMEM((11 * ECHUNK,), jnp.float32),
          pltpu.VMEM((8 * ECHUNK,), jnp.float32),
          pltpu.VMEM((8 * NCHUNK,), jnp.float32),
          pltpu.VMEM((8 * NCHUNK,), jnp.float32),
          pltpu.VMEM((8 * NCHUNK,), jnp.float32),
          pltpu.VMEM((NCHUNK,), jnp.int32),
          pltpu.VMEM((NCHUNK,), jnp.float32),
          pltpu.VMEM((NCHUNK,), jnp.float32),
          pltpu.VMEM((1024,), jnp.float32),
          pltpu.VMEM((9 * GPT,), jnp.float32),
          pltpu.VMEM((GPT,), jnp.float32),
          pltpu.SemaphoreType.DMA,
          pltpu.SemaphoreType.DMA,
      ])(_body)
  return f(h0t, post, src, dst, batchp, wflat, zeros1, ones1)


def kernel(pos, z_idx, batch, edge_index, Wm, bm, Wu, bu, Wl, bl):
  N = pos.shape[0]
  one_hot = jax.nn.one_hot(z_idx, 5, dtype=jnp.float32)
  h0 = jnp.concatenate([one_hot, jnp.zeros((N, 1), jnp.float32)], axis=1) - 0.5
  h0 = jnp.concatenate(
      [h0, jnp.ones((N, 1), jnp.float32), jnp.zeros((N, 1), jnp.float32)],
      axis=1)
  h0t = jnp.zeros((8, NPAD), jnp.float32).at[:, :N].set(h0.T).reshape(-1)
  post = jnp.zeros((3, NPAD), jnp.float32).at[:, :N].set(pos.T).reshape(-1)
  src = edge_index[0].astype(jnp.int32)
  dst = edge_index[1].astype(jnp.int32)
  batchp = jnp.full((NPAD,), N_GRAPHS, jnp.int32).at[:N].set(
      batch.astype(jnp.int32))
  wflat = jnp.zeros((1024,), jnp.float32)
  wflat = wflat.at[WM_OFF:WM_OFF + 576].set(Wm.reshape(-1))
  wflat = wflat.at[BM_OFF:BM_OFF + 24].set(bm.reshape(-1))
  wflat = wflat.at[WU_OFF:WU_OFF + 384].set(Wu.reshape(-1))
  wflat = wflat.at[BU_OFF:BU_OFF + 24].set(bu.reshape(-1))
  wflat = wflat.at[WL_OFF:WL_OFF + 8].set(Wl.reshape(-1))
  wflat = wflat.at[BL_OFF].set(bl[0])
  zeros1 = jnp.zeros((NPAD,), jnp.float32)
  ones1 = jnp.ones((NPAD,), jnp.float32)
  out = _run(h0t, post, src, dst, batchp, wflat, zeros1, ones1)
  return out.reshape(N_GRAPHS, 1)


# speedup vs baseline: 3.2507x; 1.0251x over previous
import functools
import math

import jax
import jax.numpy as jnp
from jax import lax
from jax.experimental import pallas as pl
from jax.experimental.pallas import tpu as pltpu
from jax.experimental.pallas import tpu_sc as plsc

CUTOFF = 10.0
N_RBF = 8
N_LAYERS = 3
N_GRAPHS = 1024
N_NODES = 50000
N_EDGES = 1600000

NTILES = 16
NPAD = 50176
NODES_PER_TILE = NPAD // NTILES
NCHUNK = 784
ECHUNK = 800
EQ = 400
EDGES_PER_TILE = N_EDGES // NTILES
GPAD = 1152
GPT = N_GRAPHS // NTILES
GAMMA = (CUTOFF / N_RBF) ** 2

WM_OFF = 0
BM_OFF = WM_OFF + N_LAYERS * 24 * 8
WU_OFF = BM_OFF + N_LAYERS * 8
BU_OFF = WU_OFF + N_LAYERS * 16 * 8
WL_OFF = BU_OFF + N_LAYERS * 8
BL_OFF = WL_OFF + 8


def _silu(x):
  return x / (1.0 + jnp.exp(-x))


def _body(h0_hbm, pos_hbm, src_hbm, dst_hbm, batch_hbm, w_hbm, z_hbm, one_hbm,
          out_hbm,
          h_sh, pos_sh, agg_sh, hsum_sh,
          sidx_v, didx_v, gs, gd, mc, hc, ac, ho, bidx_v, ocol, zv, w_v, rc,
          out_v, sem1, sem2):
  tid = lax.axis_index("s")
  nb_t = tid * NODES_PER_TILE

  def hj(ref, j):
    return ref.at[pl.ds(j * NPAD, NPAD)]

  stage = hc.at[pl.ds(0, NODES_PER_TILE)]
  for j in range(8):
    sl = pl.ds(j * NPAD + nb_t, NODES_PER_TILE)
    pltpu.sync_copy(h0_hbm.at[sl], stage)
    pltpu.sync_copy(stage, h_sh.at[sl])
  for a in range(3):
    sl = pl.ds(a * NPAD + nb_t, NODES_PER_TILE)
    pltpu.sync_copy(pos_hbm.at[sl], stage)
    pltpu.sync_copy(stage, pos_sh.at[sl])
  pltpu.sync_copy(z_hbm.at[pl.ds(0, NCHUNK)], zv)
  for j in range(8):
    for c in range(NODES_PER_TILE // NCHUNK):
      pltpu.sync_copy(zv, agg_sh.at[pl.ds(j * NPAD + nb_t + c * NCHUNK,
                                          NCHUNK)])
  gz = GPAD // NTILES
  for j in range(9):
    pltpu.sync_copy(zv.at[pl.ds(0, gz)],
                    hsum_sh.at[pl.ds(j * GPAD + tid * gz, gz)])
  pltpu.sync_copy(w_hbm, w_v)
  pltpu.sync_copy(one_hbm.at[pl.ds(0, NCHUNK)], ocol)
  plsc.subcore_barrier()

  centers = [CUTOFF * k / (N_RBF - 1) for k in range(N_RBF)]

  def _wscal(off):
    return w_v[pl.ds((off // 16) * 16, 16)][off % 16]

  for l in range(N_LAYERS):
    wm = [_wscal(WM_OFF + l * 192 + i) for i in range(192)]
    bm_s = [_wscal(BM_OFF + l * 8 + j) for j in range(8)]
    wu = [_wscal(WU_OFF + l * 128 + i) for i in range(128)]
    bu_s = [_wscal(BU_OFF + l * 8 + j) for j in range(8)]

    def edge_chunk(c, _):
      base = tid * EDGES_PER_TILE + c * ECHUNK
      pltpu.sync_copy(src_hbm.at[pl.ds(base, ECHUNK)], sidx_v)
      pltpu.sync_copy(dst_hbm.at[pl.ds(base, ECHUNK)], didx_v)

      def issue_q(q, sem):
        ids = sidx_v.at[pl.ds(q * EQ, EQ)]
        idd = didx_v.at[pl.ds(q * EQ, EQ)]
        cps = []
        for j in range(8):
          cps.append(pltpu.async_copy(
              hj(h_sh, j).at[ids], gs.at[pl.ds(j * ECHUNK + q * EQ, EQ)], sem))
          cps.append(pltpu.async_copy(
              hj(h_sh, j).at[idd], gd.at[pl.ds(j * ECHUNK + q * EQ, EQ)], sem))
        for a in range(3):
          cps.append(pltpu.async_copy(
              pos_sh.at[pl.ds(a * NPAD, NPAD)].at[ids],
              gs.at[pl.ds((8 + a) * ECHUNK + q * EQ, EQ)], sem))
          cps.append(pltpu.async_copy(
              pos_sh.at[pl.ds(a * NPAD, NPAD)].at[idd],
              gd.at[pl.ds((8 + a) * ECHUNK + q * EQ, EQ)], sem))
        return cps

      def group(g, _):
        o = g * 16
        hs = [gs[pl.ds(j * ECHUNK + o, 16)] for j in range(8)]
        hd = [gd[pl.ds(j * ECHUNK + o, 16)] for j in range(8)]
        rel = [gs[pl.ds((8 + a) * ECHUNK + o, 16)]
               - gd[pl.ds((8 + a) * ECHUNK + o, 16)] for a in range(3)]
        r2 = rel[0] * rel[0] + rel[1] * rel[1] + rel[2] * rel[2] + 1e-12
        i = lax.bitcast_convert_type(r2, jnp.int32)
        i = jnp.int32(0x5F3759DF) - lax.shift_right_logical(i, 1)
        y = lax.bitcast_convert_type(i, jnp.float32)
        half = 0.5 * r2
        for _n in range(3):
          y = y * (1.5 - half * y * y)
        d = r2 * y
        x = jnp.minimum(jnp.maximum(d * (1.0 / CUTOFF), 0.0), 1.0)
        u = (0.5 - x) * math.pi
        u2 = u * u
        s = u * (1.0 + u2 * (-1.0 / 6.0 + u2 * (1.0 / 120.0 + u2 * (
            -1.0 / 5040.0 + u2 * (1.0 / 362880.0 + u2 * (-1.0 / 39916800.0))))))
        env = 0.5 * (s + 1.0)
        rbf = []
        for k in range(N_RBF):
          t = d - centers[k]
          rbf.append(jnp.exp(t * t * (-1.0 / GAMMA)) * env)
        feats = hs + hd + rbf
        for j in range(8):
          acc = feats[0] * wm[j]
          for k in range(1, 24):
            acc = acc + feats[k] * wm[k * 8 + j]
          mc[pl.ds(j * ECHUNK + o, 16)] = _silu(acc + bm_s[j])
        return 0

      nq = ECHUNK // EQ
      sems = [sem1, sem2]
      pend = {0: issue_q(0, sems[0])}
      for q in range(nq):
        if q + 1 < nq:
          pend[q + 1] = issue_q(q + 1, sems[(q + 1) % 2])
        for cp in pend.pop(q):
          cp.wait()
        lax.fori_loop(q * (EQ // 16), (q + 1) * (EQ // 16), group, 0)
      scs = [pltpu.async_copy(mc.at[pl.ds(j * ECHUNK, ECHUNK)],
                              hj(agg_sh, j).at[didx_v], sem2, add=True)
             for j in range(8)]
      for cp in scs:
        cp.wait()
      return 0

    lax.fori_loop(0, EDGES_PER_TILE // ECHUNK, edge_chunk, 0)
    plsc.subcore_barrier()

    last = l == N_LAYERS - 1

    def node_chunk(c, _):
      nbase = nb_t + c * NCHUNK
      cps = []
      for j in range(8):
        cps.append(pltpu.async_copy(h_sh.at[pl.ds(j * NPAD + nbase, NCHUNK)],
                                    hc.at[pl.ds(j * NCHUNK, NCHUNK)], sem1))
        cps.append(pltpu.async_copy(agg_sh.at[pl.ds(j * NPAD + nbase, NCHUNK)],
                                    ac.at[pl.ds(j * NCHUNK, NCHUNK)], sem2))
      for cp in cps:
        cp.wait()

      def group(g, _):
        o = g * 16
        hv = [hc[pl.ds(j * NCHUNK + o, 16)] for j in range(8)]
        av = [ac[pl.ds(j * NCHUNK + o, 16)] for j in range(8)]
        for j in range(8):
          acc = hv[0] * wu[j]
          for k in range(1, 8):
            acc = acc + hv[k] * wu[k * 8 + j]
          for k in range(8):
            acc = acc + av[k] * wu[(8 + k) * 8 + j]
          ho[pl.ds(j * NCHUNK + o, 16)] = hv[j] + _silu(acc + bu_s[j])
        return 0

      lax.fori_loop(0, NCHUNK // 16, group, 0)
      if not last:
        for j in range(8):
          pltpu.sync_copy(ho.at[pl.ds(j * NCHUNK, NCHUNK)],
                          h_sh.at[pl.ds(j * NPAD + nbase, NCHUNK)])
          pltpu.sync_copy(zv, agg_sh.at[pl.ds(j * NPAD + nbase, NCHUNK)])
      else:
        pltpu.sync_copy(batch_hbm.at[pl.ds(nbase, NCHUNK)], bidx_v)
        scs = [pltpu.async_copy(
            ho.at[pl.ds(j * NCHUNK, NCHUNK)],
            hsum_sh.at[pl.ds(j * GPAD, GPAD)].at[bidx_v], sem1, add=True)
            for j in range(8)]
        scs.append(pltpu.async_copy(
            ocol, hsum_sh.at[pl.ds(8 * GPAD, GPAD)].at[bidx_v], sem1,
            add=True))
        for cp in scs:
          cp.wait()
      return 0

    lax.fori_loop(0, NODES_PER_TILE // NCHUNK, node_chunk, 0)
    plsc.subcore_barrier()

  for j in range(9):
    pltpu.sync_copy(hsum_sh.at[pl.ds(j * GPAD + tid * GPT, GPT)],
                    rc.at[pl.ds(j * GPT, GPT)])
  wl = [_wscal(WL_OFF + j) for j in range(8)]
  bl_s = _wscal(BL_OFF)

  def rgroup(g, _):
    o = g * 16
    acc = rc[pl.ds(8 * GPT + o, 16)] * bl_s
    for j in range(8):
      acc = acc + rc[pl.ds(j * GPT + o, 16)] * wl[j]
    out_v[pl.ds(o, 16)] = acc
    return 0

  lax.fori_loop(0, GPT // 16, rgroup, 0)
  pltpu.sync_copy(out_v, out_hbm.at[pl.ds(tid * GPT, GPT)])


@jax.jit
def _run(h0t, post, src, dst, batchp, wflat, zeros1, ones1):
  mesh = plsc.VectorSubcoreMesh(core_axis_name="c", subcore_axis_name="s",
                                num_cores=1)
  f = functools.partial(
      pl.kernel, mesh=mesh,
      out_type=jax.ShapeDtypeStruct((N_GRAPHS,), jnp.float32),
      scratch_types=[
          pltpu.VMEM_SHARED((8 * NPAD,), jnp.float32),
          pltpu.VMEM_SHARED((3 * NPAD,), jnp.float32),
          pltpu.VMEM_SHARED((8 * NPAD,), jnp.float32),
          pltpu.VMEM_SHARED((9 * GPAD,), jnp.float32),
          pltpu.VMEM((ECHUNK,), jnp.int32),
          pltpu.VMEM((ECHUNK,), jnp.int32),
          pltpu.VMEM((11 * ECHUNK,), jnp.float32),
          pltpu.VMEM((11 * ECHUNK,), jnp.float32),
          pltpu.VMEM((8 * ECHUNK,), jnp.float32),
          pltpu.VMEM((8 * NCHUNK,), jnp.float32),
          pltpu.VMEM((8 * NCHUNK,), jnp.float32),
          pltpu.VMEM((8 * NCHUNK,), jnp.float32),
          pltpu.VMEM((NCHUNK,), jnp.int32),
          pltpu.VMEM((NCHUNK,), jnp.float32),
          pltpu.VMEM((NCHUNK,), jnp.float32),
          pltpu.VMEM((1024,), jnp.float32),
          pltpu.VMEM((9 * GPT,), jnp.float32),
          pltpu.VMEM((GPT,), jnp.float32),
          pltpu.SemaphoreType.DMA,
          pltpu.SemaphoreType.DMA,
      ])(_body)
  return f(h0t, post, src, dst, batchp, wflat, zeros1, ones1)


def kernel(pos, z_idx, batch, edge_index, Wm, bm, Wu, bu, Wl, bl):
  N = pos.shape[0]
  one_hot = jax.nn.one_hot(z_idx, 5, dtype=jnp.float32)
  h0 = jnp.concatenate([one_hot, jnp.zeros((N, 1), jnp.float32)], axis=1) - 0.5
  h0 = jnp.concatenate(
      [h0, jnp.ones((N, 1), jnp.float32), jnp.zeros((N, 1), jnp.float32)],
      axis=1)
  h0t = jnp.zeros((8, NPAD), jnp.float32).at[:, :N].set(h0.T).reshape(-1)
  post = jnp.zeros((3, NPAD), jnp.float32).at[:, :N].set(pos.T).reshape(-1)
  src = edge_index[0].astype(jnp.int32)
  dst = edge_index[1].astype(jnp.int32)
  batchp = jnp.full((NPAD,), N_GRAPHS, jnp.int32).at[:N].set(
      batch.astype(jnp.int32))
  wflat = jnp.zeros((1024,), jnp.float32)
  wflat = wflat.at[WM_OFF:WM_OFF + 576].set(Wm.reshape(-1))
  wflat = wflat.at[BM_OFF:BM_OFF + 24].set(bm.reshape(-1))
  wflat = wflat.at[WU_OFF:WU_OFF + 384].set(Wu.reshape(-1))
  wflat = wflat.at[BU_OFF:BU_OFF + 24].set(bu.reshape(-1))
  wflat = wflat.at[WL_OFF:WL_OFF + 8].set(Wl.reshape(-1))
  wflat = wflat.at[BL_OFF].set(bl[0])
  zeros1 = jnp.zeros((NPAD,), jnp.float32)
  ones1 = jnp.ones((NPAD,), jnp.float32)
  out = _run(h0t, post, src, dst, batchp, wflat, zeros1, ones1)
  return out.reshape(N_GRAPHS, 1)
